# Initial kernel scaffold; baseline (speedup 1.0000x reference)
import jax
import jax.numpy as jnp
from jax.experimental import pallas as pl


def kernel(positions, cos, sin):
    raise NotImplementedError("write your pallas kernel here")


# speedup vs baseline: 4.8768x; 4.8768x over previous
import functools

import jax
import jax.numpy as jnp
from jax import lax
from jax.experimental import pallas as pl
from jax.experimental.pallas import tpu as pltpu
from jax.experimental.pallas import tpu_sc as plsc

D = 128
NC = 2
NS = 16
NW = NC * NS
CH = 128


@functools.lru_cache(maxsize=None)
def _make_kernel(B):
    assert B % (NW * CH) == 0
    n_chunks = B // (NW * CH)
    mesh = plsc.VectorSubcoreMesh(core_axis_name="c", subcore_axis_name="s")

    @functools.partial(
        pl.kernel,
        mesh=mesh,
        out_type=(
            jax.ShapeDtypeStruct((B, D), jnp.float32),
            jax.ShapeDtypeStruct((B, D), jnp.float32),
        ),
        scratch_types=[
            pltpu.VMEM((n_chunks, CH), jnp.int32),
            pltpu.VMEM((CH, D), jnp.float32),
            pltpu.VMEM((CH, D), jnp.float32),
            pltpu.SemaphoreType.DMA,
            pltpu.SemaphoreType.DMA,
        ],
    )
    def body(pos_hbm, cos_hbm, sin_hbm, cos_out, sin_out,
             idx_v, cos_v, sin_v, sem_c, sem_s):
        wid = lax.axis_index("s") * NC + lax.axis_index("c")
        pltpu.sync_copy(pos_hbm.at[wid], idx_v)
        for c in range(n_chunks):
            g_cos = pltpu.async_copy(cos_hbm.at[idx_v.at[c]], cos_v, sem_c)
            g_sin = pltpu.async_copy(sin_hbm.at[idx_v.at[c]], sin_v, sem_s)
            base = wid * (n_chunks * CH) + c * CH
            g_cos.wait()
            pltpu.sync_copy(cos_v, cos_out.at[pl.ds(base, CH)])
            g_sin.wait()
            pltpu.sync_copy(sin_v, sin_out.at[pl.ds(base, CH)])

    return body


def kernel(positions, cos, sin):
    B = positions.size
    pos = positions.reshape(NW, B // (NW * CH), CH).astype(jnp.int32)
    cos_out, sin_out = _make_kernel(B)(pos, cos, sin)
    return (cos_out, sin_out)


# speedup vs baseline: 5.1332x; 1.0526x over previous
import functools

import jax
import jax.numpy as jnp
from jax import lax
from jax.experimental import pallas as pl
from jax.experimental.pallas import tpu as pltpu
from jax.experimental.pallas import tpu_sc as plsc

D = 128
NC = 2
NS = 16
NW = NC * NS
CH = 128


@functools.lru_cache(maxsize=None)
def _make_kernel(B):
    assert B % (NW * CH) == 0
    n_chunks = B // (NW * CH)
    mesh = plsc.VectorSubcoreMesh(core_axis_name="c", subcore_axis_name="s")

    @functools.partial(
        pl.kernel,
        mesh=mesh,
        out_type=(
            jax.ShapeDtypeStruct((B, D), jnp.float32),
            jax.ShapeDtypeStruct((B, D), jnp.float32),
        ),
        scratch_types=[
            pltpu.VMEM((n_chunks, CH), jnp.int32),
            pltpu.VMEM((2, CH, D), jnp.float32),
            pltpu.VMEM((2, CH, D), jnp.float32),
            pltpu.SemaphoreType.DMA,
            pltpu.SemaphoreType.DMA,
            pltpu.SemaphoreType.DMA,
            pltpu.SemaphoreType.DMA,
            pltpu.SemaphoreType.DMA,
            pltpu.SemaphoreType.DMA,
            pltpu.SemaphoreType.DMA,
            pltpu.SemaphoreType.DMA,
        ],
    )
    def body(pos_hbm, cos_hbm, sin_hbm, cos_out, sin_out,
             idx_v, cos_v, sin_v,
             gc0, gc1, gs0, gs1, sc0, sc1, ss0, ss1):
        gc, gs, sc, ss = [gc0, gc1], [gs0, gs1], [sc0, sc1], [ss0, ss1]
        wid = lax.axis_index("s") * NC + lax.axis_index("c")
        pltpu.sync_copy(pos_hbm.at[wid], idx_v)
        gd = [None, None]
        sd = [None, None]

        def start_gather(c):
            b = c % 2
            gd[b] = (pltpu.async_copy(cos_hbm.at[idx_v.at[c]], cos_v.at[b], gc[b]),
                     pltpu.async_copy(sin_hbm.at[idx_v.at[c]], sin_v.at[b], gs[b]))

        start_gather(0)
        for c in range(n_chunks):
            b = c % 2
            for d in gd[b]:
                d.wait()
            if c + 1 < n_chunks:
                nb = (c + 1) % 2
                if sd[nb] is not None:
                    for d in sd[nb]:
                        d.wait()
                    sd[nb] = None
                start_gather(c + 1)
            base = wid * (n_chunks * CH) + c * CH
            sd[b] = (pltpu.async_copy(cos_v.at[b], cos_out.at[pl.ds(base, CH)], sc[b]),
                     pltpu.async_copy(sin_v.at[b], sin_out.at[pl.ds(base, CH)], ss[b]))
        for b in range(2):
            if sd[b] is not None:
                for d in sd[b]:
                    d.wait()

    return body


def kernel(positions, cos, sin):
    B = positions.size
    pos = positions.reshape(NW, B // (NW * CH), CH).astype(jnp.int32)
    cos_out, sin_out = _make_kernel(B)(pos, cos, sin)
    return (cos_out, sin_out)


# speedup vs baseline: 5.1672x; 1.0066x over previous
import functools

import jax
import jax.numpy as jnp
from jax import lax
from jax.experimental import pallas as pl
from jax.experimental.pallas import tpu as pltpu
from jax.experimental.pallas import tpu_sc as plsc

D = 128
NC = 2
NS = 16
NW = NC * NS
CH = 128
NBUF = 3


@functools.lru_cache(maxsize=None)
def _make_kernel(rows, cols):
    B = rows * cols
    assert B % (NW * CH) == 0
    per_w = B // NW
    n_chunks = per_w // CH
    w_per_row = cols // per_w
    mesh = plsc.VectorSubcoreMesh(core_axis_name="c", subcore_axis_name="s")

    @functools.partial(
        pl.kernel,
        mesh=mesh,
        out_type=(
            jax.ShapeDtypeStruct((B, D), jnp.float32),
            jax.ShapeDtypeStruct((B, D), jnp.float32),
        ),
        scratch_types=[
            pltpu.VMEM((per_w,), jnp.int32),
            pltpu.VMEM((NBUF, CH, D), jnp.float32),
            pltpu.VMEM((NBUF, CH, D), jnp.float32),
        ]
        + [pltpu.SemaphoreType.DMA] * (4 * NBUF),
    )
    def body(pos_hbm, cos_hbm, sin_hbm, cos_out, sin_out,
             idx_v, cos_v, sin_v, *sems):
        gc, gs, sc, ss = (list(sems[i * NBUF:(i + 1) * NBUF]) for i in range(4))
        wid = lax.axis_index("s") * NC + lax.axis_index("c")
        prow = wid // w_per_row
        pcol = (wid % w_per_row) * per_w
        pltpu.sync_copy(pos_hbm.at[prow, pl.ds(pcol, per_w)], idx_v)
        gd = [None] * NBUF
        sd = [None] * NBUF

        def start_gather(c):
            b = c % NBUF
            ix = idx_v.at[pl.ds(c * CH, CH)]
            gd[b] = (pltpu.async_copy(cos_hbm.at[ix], cos_v.at[b], gc[b]),
                     pltpu.async_copy(sin_hbm.at[ix], sin_v.at[b], gs[b]))

        for c in range(min(2, n_chunks)):
            start_gather(c)
        for c in range(n_chunks):
            b = c % NBUF
            for d in gd[b]:
                d.wait()
            base = wid * per_w + c * CH
            sd[b] = (pltpu.async_copy(cos_v.at[b], cos_out.at[pl.ds(base, CH)], sc[b]),
                     pltpu.async_copy(sin_v.at[b], sin_out.at[pl.ds(base, CH)], ss[b]))
            nxt = c + 2
            if nxt < n_chunks:
                nb = nxt % NBUF
                if sd[nb] is not None:
                    for d in sd[nb]:
                        d.wait()
                    sd[nb] = None
                start_gather(nxt)
        for b in range(NBUF):
            if sd[b] is not None:
                for d in sd[b]:
                    d.wait()

    return body


def kernel(positions, cos, sin):
    pos = positions.astype(jnp.int32)
    cos_out, sin_out = _make_kernel(pos.shape[0], pos.shape[1])(pos, cos, sin)
    return (cos_out, sin_out)
